# baseline (speedup 1.0000x reference)
import functools

import jax
import jax.numpy as jnp
from jax import lax
from jax.experimental import pallas as pl
from jax.experimental.pallas import tpu as pltpu
from jax.experimental.pallas import tpu_sc as plsc

B = 16384
D = 64
NLBL = 64

NC, NS = 2, 16
NW = NC * NS
BPW = B // NW
CH = 128
NCH = BPW // CH

_mesh = plsc.VectorSubcoreMesh(core_axis_name="c", subcore_axis_name="s")


@functools.partial(
    pl.kernel,
    mesh=_mesh,
    out_type=jax.ShapeDtypeStruct((B, D), jnp.float32),
    scratch_types=[
        pltpu.VMEM((NCH, CH), jnp.int32),
        pltpu.VMEM((BPW, D), jnp.float32),
        pltpu.SemaphoreType.DMA,
    ],
    compiler_params=pltpu.CompilerParams(use_tc_tiling_on_sc=False),
)
def _sc_gather(idx_hbm, table_hbm, out_hbm, idx_v, rows_v, sem):
    wid = lax.axis_index("s") * NC + lax.axis_index("c")
    base = wid * BPW
    pltpu.sync_copy(idx_hbm.at[wid], idx_v)
    copies = []
    for j in range(NCH):
        copies.append(
            pltpu.async_copy(
                table_hbm.at[idx_v.at[j]],
                rows_v.at[pl.ds(j * CH, CH)],
                sem,
            )
        )
    for c in copies:
        c.wait()
    pltpu.sync_copy(rows_v, out_hbm.at[pl.ds(base, BPW)])


def _mm_body(g_ref, wt_ref, b_ref, o_ref):
    o_ref[...] = (
        jnp.dot(g_ref[...], wt_ref[...], preferred_element_type=jnp.float32)
        + b_ref[...]
    )


MB = 2048


def _tc_linear(g, wt, b2):
    return pl.pallas_call(
        _mm_body,
        grid=(B // MB,),
        in_specs=[
            pl.BlockSpec((MB, D), lambda i: (i, 0)),
            pl.BlockSpec((D, NLBL), lambda i: (0, 0)),
            pl.BlockSpec((1, NLBL), lambda i: (0, 0)),
        ],
        out_specs=pl.BlockSpec((MB, NLBL), lambda i: (i, 0)),
        out_shape=jax.ShapeDtypeStruct((B, NLBL), jnp.float32),
    )(g, wt, b2)


def kernel(node_seq, table, W, b):
    idx3 = node_seq.astype(jnp.int32).reshape(NW, NCH, CH)
    g = _sc_gather(idx3, table)
    return _tc_linear(g, W.T, b.reshape(1, NLBL))
